# Initial kernel scaffold; baseline (speedup 1.0000x reference)
import jax
import jax.numpy as jnp
from jax.experimental import pallas as pl


def kernel(x, table1, table2, Wl, bl, W1, b1, W2, b2, W3, b3, W4, b4):
    raise NotImplementedError("write your pallas kernel here")


# speedup vs baseline: 2.9366x; 2.9366x over previous
import functools

import jax
import jax.numpy as jnp
from jax import lax
from jax.experimental import pallas as pl
from jax.experimental.pallas import tpu as pltpu
from jax.experimental.pallas import tpu_sc as plsc

HS = 2 ** 19
MASK = HS - 1
NPTS = 200
C1 = 2654435761
C2 = 805459861
NL1 = 5
NL2 = 6
LANES = 16


def _hash_corner(ix, iy_m, iz_m):
    h = ix.astype(jnp.uint32) ^ iy_m ^ iz_m
    return (h & jnp.uint32(MASK)).astype(jnp.int32)


def _coords(ox, oy, dx, dy, tv):
    px = jnp.clip(ox + tv * dx, 0.0, 1.0)
    py = jnp.clip(oy + tv * dy, 0.0, 1.0)
    return px, py, tv


def _cell(p, res):
    pos = p * res
    p0 = pos.astype(jnp.int32)
    frac = pos - p0.astype(jnp.float32)
    return p0, frac


def _corner_weights(fx, fy, fz):
    wx0, wy0, wz0 = 1.0 - fx, 1.0 - fy, 1.0 - fz
    ws = []
    for wx in (wx0, fx):
        for wy in (wy0, fy):
            for wz in (wz0, fz):
                ws.append((wx * wy) * wz)
    return ws


def _corner_hashes(px, py, pz, res):
    x0, fx = _cell(px, res)
    y0, fy = _cell(py, res)
    z0, fz = _cell(pz, res)
    ym0 = y0.astype(jnp.uint32) * jnp.uint32(C1)
    ym1 = (y0 + 1).astype(jnp.uint32) * jnp.uint32(C1)
    zm0 = z0.astype(jnp.uint32) * jnp.uint32(C2)
    zm1 = (z0 + 1).astype(jnp.uint32) * jnp.uint32(C2)
    hs = []
    for xx in (x0, x0 + 1):
        for ym in (ym0, ym1):
            for zm in (zm0, zm1):
                hs.append(_hash_corner(xx, ym, zm))
    return hs, fx, fy, fz


def _enc1_kernel_body(n_rays, rpw, num_cores,
                      xf_hbm, wlb_hbm, blb_hbm, t1_hbm, z_hbm,
                      xv, wlv, blv, idxb, valb, zbuf):
    wid = lax.axis_index("s") * num_cores + lax.axis_index("c")
    base = wid * rpw
    for r in range(4):
        pltpu.sync_copy(xf_hbm.at[pl.ds(r * n_rays + base, rpw)],
                        xv.at[pl.ds(r * rpw, rpw)])
    pltpu.sync_copy(wlb_hbm, wlv)
    pltpu.sync_copy(blb_hbm, blv)
    iota = lax.iota(jnp.int32, LANES)
    n_groups = rpw // LANES

    @pl.loop(0, n_groups)
    def _group(g):
        g16 = g * LANES
        ox = xv[pl.ds(0 * rpw + g16, LANES)]
        oy = xv[pl.ds(1 * rpw + g16, LANES)]
        dx = xv[pl.ds(2 * rpw + g16, LANES)]
        dy = xv[pl.ds(3 * rpw + g16, LANES)]
        blvec = blv[...]
        zrow = iota * NPTS

        @pl.loop(0, NPTS)
        def _point(j):
            tv = jnp.full((LANES,), j.astype(jnp.float32)) / 199.0
            px, py, pz = _coords(ox, oy, dx, dy, tv)
            fracs = []
            for l in range(NL1):
                hs, fx, fy, fz = _corner_hashes(px, py, pz, float(8 << l))
                fracs.append((fx, fy, fz))
                for c in range(8):
                    idxb[pl.ds((l * 8 + c) * LANES, LANES)] = hs[c] + (l * HS)
            pltpu.sync_copy(t1_hbm.at[idxb], valb)
            z = blvec
            for l in range(NL1):
                fx, fy, fz = fracs[l]
                ws = _corner_weights(fx, fy, fz)
                s = ws[0] * valb[pl.ds((l * 8) * LANES, LANES)]
                for c in range(1, 8):
                    s = s + ws[c] * valb[pl.ds((l * 8 + c) * LANES, LANES)]
                z = z + s * wlv[pl.ds(l * LANES, LANES)]
            plsc.store_scatter(zbuf, [zrow + j], z)

        pltpu.sync_copy(zbuf, z_hbm.at[pl.ds((base + g16) * NPTS,
                                             LANES * NPTS)])


def _enc1_logits(xf, t1f, wlb, blb, n_rays):
    info = plsc.get_sparse_core_info()
    nw = info.num_cores * info.num_subcores
    rpw = n_rays // nw
    mesh = plsc.VectorSubcoreMesh(core_axis_name="c", subcore_axis_name="s")
    body = functools.partial(_enc1_kernel_body, n_rays, rpw, info.num_cores)
    return pl.kernel(
        body,
        out_type=jax.ShapeDtypeStruct((n_rays * NPTS,), jnp.float32),
        mesh=mesh,
        compiler_params=pltpu.CompilerParams(needs_layout_passes=False),
        scratch_types=[
            pltpu.VMEM((4 * rpw,), jnp.float32),
            pltpu.VMEM((NL1 * LANES,), jnp.float32),
            pltpu.VMEM((LANES,), jnp.float32),
            pltpu.VMEM((NL1 * 8 * LANES,), jnp.int32),
            pltpu.VMEM((NL1 * 8 * LANES,), jnp.float32),
            pltpu.VMEM((LANES * NPTS,), jnp.float32),
        ],
    )(xf, wlb, blb, t1f)


def _label_kernel(z_ref, lab_ref, hits_ref, idx_ref):
    z = z_ref[...]
    lab = 1.0 / (1.0 + jnp.exp(-z))
    lab_ref[...] = lab
    hits_ref[...] = jnp.max(lab, axis=1, keepdims=True)
    cond = lab > 0.5
    jmat = lax.broadcasted_iota(jnp.int32, z.shape, 1)
    masked = jnp.where(cond, jmat, NPTS)
    first = jnp.min(masked, axis=1, keepdims=True)
    idx_ref[...] = jnp.where(first == NPTS, 0, first)


def _labels(z, n_rays):
    blk = 256
    grid = n_rays // blk
    return pl.pallas_call(
        _label_kernel,
        grid=(grid,),
        in_specs=[pl.BlockSpec((blk, NPTS), lambda i: (i, 0))],
        out_specs=[
            pl.BlockSpec((blk, NPTS), lambda i: (i, 0)),
            pl.BlockSpec((blk, 1), lambda i: (i, 0)),
            pl.BlockSpec((blk, 1), lambda i: (i, 0)),
        ],
        out_shape=[
            jax.ShapeDtypeStruct((n_rays, NPTS), jnp.float32),
            jax.ShapeDtypeStruct((n_rays, 1), jnp.float32),
            jax.ShapeDtypeStruct((n_rays, 1), jnp.int32),
        ],
    )(z)


def _enc2_kernel_body(n_rays, rpw, num_cores,
                      xf_hbm, idx_hbm, t2_hbm, feat_hbm,
                      xv, iv, idxb, valb, fbuf):
    wid = lax.axis_index("s") * num_cores + lax.axis_index("c")
    base = wid * rpw
    for r in range(4):
        pltpu.sync_copy(xf_hbm.at[pl.ds(r * n_rays + base, rpw)],
                        xv.at[pl.ds(r * rpw, rpw)])
    pltpu.sync_copy(idx_hbm.at[pl.ds(base, rpw)], iv)
    iota = lax.iota(jnp.int32, LANES)
    n_groups = rpw // LANES
    nfeat = 5 * NL2 * 4

    @pl.loop(0, n_groups)
    def _group(g):
        g16 = g * LANES
        ox = xv[pl.ds(0 * rpw + g16, LANES)]
        oy = xv[pl.ds(1 * rpw + g16, LANES)]
        dx = xv[pl.ds(2 * rpw + g16, LANES)]
        dy = xv[pl.ds(3 * rpw + g16, LANES)]
        jfirst = iv[pl.ds(g16, LANES)]
        frow = iota * nfeat

        @pl.loop(0, 5)
        def _neighbor(k):
            jk = jnp.clip(jfirst + (k - 2), 0, NPTS - 1)
            tv = jk.astype(jnp.float32) / 199.0
            px, py, pz = _coords(ox, oy, dx, dy, tv)
            fracs = []
            for l in range(NL2):
                hs, fx, fy, fz = _corner_hashes(px, py, pz, float(4 << l))
                fracs.append((fx, fy, fz))
                for c in range(8):
                    h4 = (hs[c] + l * HS) * 4
                    for f in range(4):
                        idxb[pl.ds(((l * 8 + c) * 4 + f) * LANES, LANES)] = (
                            h4 + f)
            pltpu.sync_copy(t2_hbm.at[idxb], valb)
            for l in range(NL2):
                fx, fy, fz = fracs[l]
                ws = _corner_weights(fx, fy, fz)
                for f in range(4):
                    s = ws[0] * valb[pl.ds((l * 8 * 4 + f) * LANES, LANES)]
                    for c in range(1, 8):
                        s = s + ws[c] * valb[
                            pl.ds(((l * 8 + c) * 4 + f) * LANES, LANES)]
                    col = k * (NL2 * 4) + l * 4 + f
                    plsc.store_scatter(fbuf, [frow + col], s)

        pltpu.sync_copy(fbuf, feat_hbm.at[pl.ds((base + g16) * nfeat,
                                                LANES * nfeat)])


def _enc2_feat(xf, idxf, t2f, n_rays):
    info = plsc.get_sparse_core_info()
    nw = info.num_cores * info.num_subcores
    rpw = n_rays // nw
    mesh = plsc.VectorSubcoreMesh(core_axis_name="c", subcore_axis_name="s")
    body = functools.partial(_enc2_kernel_body, n_rays, rpw, info.num_cores)
    nfeat = 5 * NL2 * 4
    return pl.kernel(
        body,
        out_type=jax.ShapeDtypeStruct((n_rays * nfeat,), jnp.float32),
        mesh=mesh,
        compiler_params=pltpu.CompilerParams(needs_layout_passes=False),
        scratch_types=[
            pltpu.VMEM((4 * rpw,), jnp.float32),
            pltpu.VMEM((rpw,), jnp.int32),
            pltpu.VMEM((NL2 * 8 * 4 * LANES,), jnp.int32),
            pltpu.VMEM((NL2 * 8 * 4 * LANES,), jnp.float32),
            pltpu.VMEM((LANES * nfeat,), jnp.float32),
        ],
    )(xf, idxf, t2f)


def _mlp_kernel(f_ref, w1_ref, b1_ref, w2_ref, b2_ref, w3_ref, b3_ref,
                w4_ref, b4_ref, out_ref):
    h = jnp.maximum(
        jnp.dot(f_ref[...], w1_ref[...], preferred_element_type=jnp.float32)
        + b1_ref[...], 0.0)
    h = jnp.maximum(
        jnp.dot(h, w2_ref[...], preferred_element_type=jnp.float32)
        + b2_ref[...], 0.0)
    h = jnp.maximum(
        jnp.dot(h, w3_ref[...], preferred_element_type=jnp.float32)
        + b3_ref[...], 0.0)
    out_ref[...] = (
        jnp.dot(h, w4_ref[...], preferred_element_type=jnp.float32)
        + b4_ref[...])


def _mlp(feat, W1, b1, W2, b2, W3, b3, W4p, b4p, n_rays):
    blk = 256
    grid = n_rays // blk
    full = lambda shape: pl.BlockSpec(shape, lambda i: (0, 0))
    return pl.pallas_call(
        _mlp_kernel,
        grid=(grid,),
        in_specs=[
            pl.BlockSpec((blk, 120), lambda i: (i, 0)),
            full((120, 64)), full((1, 64)),
            full((64, 64)), full((1, 64)),
            full((64, 64)), full((1, 64)),
            full((64, 8)), full((1, 8)),
        ],
        out_specs=pl.BlockSpec((blk, 8), lambda i: (i, 0)),
        out_shape=jax.ShapeDtypeStruct((n_rays, 8), jnp.float32),
    )(feat, W1, b1.reshape(1, 64), W2, b2.reshape(1, 64),
      W3, b3.reshape(1, 64), W4p, b4p)


def kernel(x, table1, table2, Wl, bl, W1, b1, W2, b2, W3, b3, W4, b4):
    n_rays = x.shape[0]
    xf = x.T.reshape(4 * n_rays)
    t1f = table1.reshape(NL1 * HS)
    t2f = table2.reshape(NL2 * HS * 4)
    wlb = jnp.broadcast_to(Wl.reshape(NL1, 1), (NL1, LANES)).reshape(-1)
    blb = jnp.broadcast_to(bl.reshape(1, 1), (1, LANES)).reshape(-1)

    z = _enc1_logits(xf, t1f, wlb, blb, n_rays).reshape(n_rays, NPTS)
    label, hits, idxf = _labels(z, n_rays)
    feat = _enc2_feat(xf, idxf.reshape(n_rays), t2f, n_rays)
    feat = feat.reshape(n_rays, 5 * NL2 * 4)

    W4p = jnp.concatenate([W4, jnp.zeros((64, 5), W4.dtype)], axis=1)
    b4p = jnp.concatenate([b4, jnp.zeros((5,), b4.dtype)]).reshape(1, 8)
    rgb8 = _mlp(feat, W1, b1, W2, b2, W3, b3, W4p, b4p, n_rays)

    return (hits, label.reshape(n_rays, NPTS, 1), rgb8[:, :3])


# speedup vs baseline: 3.3213x; 1.1310x over previous
import functools

import jax
import jax.numpy as jnp
from jax import lax
from jax.experimental import pallas as pl
from jax.experimental.pallas import tpu as pltpu
from jax.experimental.pallas import tpu_sc as plsc

HS = 2 ** 19
MASK = HS - 1
NPTS = 200
C1 = 2654435761
C2 = 805459861
NL1 = 5
NL2 = 6
LANES = 16


def _hash_corner(ix, iy_m, iz_m):
    h = ix.astype(jnp.uint32) ^ iy_m ^ iz_m
    return (h & jnp.uint32(MASK)).astype(jnp.int32)


def _coords(ox, oy, dx, dy, tv):
    px = jnp.clip(ox + tv * dx, 0.0, 1.0)
    py = jnp.clip(oy + tv * dy, 0.0, 1.0)
    return px, py, tv


def _cell(p, res):
    pos = p * res
    p0 = pos.astype(jnp.int32)
    frac = pos - p0.astype(jnp.float32)
    return p0, frac


def _corner_weights(fx, fy, fz):
    wx0, wy0, wz0 = 1.0 - fx, 1.0 - fy, 1.0 - fz
    ws = []
    for wx in (wx0, fx):
        for wy in (wy0, fy):
            for wz in (wz0, fz):
                ws.append((wx * wy) * wz)
    return ws


def _corner_hashes(px, py, pz, res):
    x0, fx = _cell(px, res)
    y0, fy = _cell(py, res)
    z0, fz = _cell(pz, res)
    ym0 = y0.astype(jnp.uint32) * jnp.uint32(C1)
    ym1 = (y0 + 1).astype(jnp.uint32) * jnp.uint32(C1)
    zm0 = z0.astype(jnp.uint32) * jnp.uint32(C2)
    zm1 = (z0 + 1).astype(jnp.uint32) * jnp.uint32(C2)
    hs = []
    for xx in (x0, x0 + 1):
        for ym in (ym0, ym1):
            for zm in (zm0, zm1):
                hs.append(_hash_corner(xx, ym, zm))
    return hs, fx, fy, fz


JB = 4
NIDX1 = NL1 * 8 * LANES


def _enc1_kernel_body(n_rays, rpw, num_cores,
                      xf_hbm, wlb_hbm, blb_hbm, t1_hbm, z_hbm,
                      xv, wlv, blv, idxA, idxB, valA, valB, zbuf,
                      semA, semB):
    wid = lax.axis_index("s") * num_cores + lax.axis_index("c")
    base = wid * rpw
    for r in range(4):
        pltpu.sync_copy(xf_hbm.at[pl.ds(r * n_rays + base, rpw)],
                        xv.at[pl.ds(r * rpw, rpw)])
    pltpu.sync_copy(wlb_hbm, wlv)
    pltpu.sync_copy(blb_hbm, blv)
    iota = lax.iota(jnp.int32, LANES)
    n_groups = rpw // LANES

    @pl.loop(0, n_groups)
    def _group(g):
        g16 = g * LANES
        ox = xv[pl.ds(0 * rpw + g16, LANES)]
        oy = xv[pl.ds(1 * rpw + g16, LANES)]
        dx = xv[pl.ds(2 * rpw + g16, LANES)]
        dy = xv[pl.ds(3 * rpw + g16, LANES)]
        blvec = blv[...]
        zrow = iota * NPTS

        def build(jb, idxb):
            for u in range(JB):
                j = jnp.asarray(jb + u, jnp.float32)
                tv = jnp.full((LANES,), 1.0, jnp.float32) * (j * (1.0 / 199.0))
                px, py, pz = _coords(ox, oy, dx, dy, tv)
                for l in range(NL1):
                    hs, _, _, _ = _corner_hashes(px, py, pz, float(8 << l))
                    for c in range(8):
                        idxb[pl.ds((u * NL1 * 8 + l * 8 + c) * LANES,
                                   LANES)] = hs[c] + (l * HS)

        def start(idxb, valb, sem):
            return pltpu.async_copy(t1_hbm.at[idxb], valb, sem)

        def wait(idxb, valb, sem):
            pltpu.make_async_copy(t1_hbm.at[idxb], valb, sem).wait()

        def compute(jb, valb):
            for u in range(JB):
                j = jb + u
                jf = jnp.asarray(j, jnp.float32)
                tv = jnp.full((LANES,), 1.0, jnp.float32) * (jf * (1.0 / 199.0))
                px, py, pz = _coords(ox, oy, dx, dy, tv)
                z = blvec
                for l in range(NL1):
                    _, fx = _cell(px, float(8 << l))
                    _, fy = _cell(py, float(8 << l))
                    _, fz = _cell(pz, float(8 << l))
                    ws = _corner_weights(fx, fy, fz)
                    s = ws[0] * valb[pl.ds((u * NL1 * 8 + l * 8) * LANES,
                                           LANES)]
                    for c in range(1, 8):
                        s = s + ws[c] * valb[
                            pl.ds((u * NL1 * 8 + l * 8 + c) * LANES, LANES)]
                    z = z + s * wlv[pl.ds(l * LANES, LANES)]
                plsc.store_scatter(zbuf, [zrow + j], z)

        build(0, idxA)
        start(idxA, valA, semA)
        n_iter = NPTS // (2 * JB)

        @pl.loop(0, n_iter)
        def _pair(i):
            jb = i * (2 * JB)
            build(jb + JB, idxB)
            start(idxB, valB, semB)
            wait(idxA, valA, semA)
            compute(jb, valA)

            @pl.when(i < n_iter - 1)
            def _prefetch():
                build(jb + 2 * JB, idxA)
                start(idxA, valA, semA)

            wait(idxB, valB, semB)
            compute(jb + JB, valB)

        pltpu.sync_copy(zbuf, z_hbm.at[pl.ds((base + g16) * NPTS,
                                             LANES * NPTS)])


def _enc1_logits(xf, t1f, wlb, blb, n_rays):
    info = plsc.get_sparse_core_info()
    nw = info.num_cores * info.num_subcores
    rpw = n_rays // nw
    mesh = plsc.VectorSubcoreMesh(core_axis_name="c", subcore_axis_name="s")
    body = functools.partial(_enc1_kernel_body, n_rays, rpw, info.num_cores)
    return pl.kernel(
        body,
        out_type=jax.ShapeDtypeStruct((n_rays * NPTS,), jnp.float32),
        mesh=mesh,
        compiler_params=pltpu.CompilerParams(needs_layout_passes=False),
        scratch_types=[
            pltpu.VMEM((4 * rpw,), jnp.float32),
            pltpu.VMEM((NL1 * LANES,), jnp.float32),
            pltpu.VMEM((LANES,), jnp.float32),
            pltpu.VMEM((JB * NIDX1,), jnp.int32),
            pltpu.VMEM((JB * NIDX1,), jnp.int32),
            pltpu.VMEM((JB * NIDX1,), jnp.float32),
            pltpu.VMEM((JB * NIDX1,), jnp.float32),
            pltpu.VMEM((LANES * NPTS,), jnp.float32),
            pltpu.SemaphoreType.DMA,
            pltpu.SemaphoreType.DMA,
        ],
    )(xf, wlb, blb, t1f)


def _label_kernel(z_ref, lab_ref, hits_ref, idx_ref):
    z = z_ref[...]
    lab = 1.0 / (1.0 + jnp.exp(-z))
    lab_ref[...] = lab
    hits_ref[...] = jnp.max(lab, axis=1, keepdims=True)
    cond = lab > 0.5
    jmat = lax.broadcasted_iota(jnp.int32, z.shape, 1)
    masked = jnp.where(cond, jmat, NPTS)
    first = jnp.min(masked, axis=1, keepdims=True)
    idx_ref[...] = jnp.where(first == NPTS, 0, first)


def _labels(z, n_rays):
    blk = 256
    grid = n_rays // blk
    return pl.pallas_call(
        _label_kernel,
        grid=(grid,),
        in_specs=[pl.BlockSpec((blk, NPTS), lambda i: (i, 0))],
        out_specs=[
            pl.BlockSpec((blk, NPTS), lambda i: (i, 0)),
            pl.BlockSpec((blk, 1), lambda i: (i, 0)),
            pl.BlockSpec((blk, 1), lambda i: (i, 0)),
        ],
        out_shape=[
            jax.ShapeDtypeStruct((n_rays, NPTS), jnp.float32),
            jax.ShapeDtypeStruct((n_rays, 1), jnp.float32),
            jax.ShapeDtypeStruct((n_rays, 1), jnp.int32),
        ],
    )(z)


def _enc2_kernel_body(n_rays, rpw, num_cores,
                      xf_hbm, idx_hbm, t2_hbm, feat_hbm,
                      xv, iv, idxb, valb, fbuf):
    wid = lax.axis_index("s") * num_cores + lax.axis_index("c")
    base = wid * rpw
    for r in range(4):
        pltpu.sync_copy(xf_hbm.at[pl.ds(r * n_rays + base, rpw)],
                        xv.at[pl.ds(r * rpw, rpw)])
    pltpu.sync_copy(idx_hbm.at[pl.ds(base, rpw)], iv)
    iota = lax.iota(jnp.int32, LANES)
    n_groups = rpw // LANES
    nfeat = 5 * NL2 * 4

    @pl.loop(0, n_groups)
    def _group(g):
        g16 = g * LANES
        ox = xv[pl.ds(0 * rpw + g16, LANES)]
        oy = xv[pl.ds(1 * rpw + g16, LANES)]
        dx = xv[pl.ds(2 * rpw + g16, LANES)]
        dy = xv[pl.ds(3 * rpw + g16, LANES)]
        jfirst = iv[pl.ds(g16, LANES)]
        frow = iota * nfeat

        @pl.loop(0, 5)
        def _neighbor(k):
            jk = jnp.clip(jfirst + (k - 2), 0, NPTS - 1)
            tv = jk.astype(jnp.float32) * (1.0 / 199.0)
            px, py, pz = _coords(ox, oy, dx, dy, tv)
            fracs = []
            for l in range(NL2):
                hs, fx, fy, fz = _corner_hashes(px, py, pz, float(4 << l))
                fracs.append((fx, fy, fz))
                for c in range(8):
                    entry4 = (hs[c] + l * HS) * 4
                    for f in range(4):
                        idxb[pl.ds(((l * 8 + c) * 4 + f) * LANES,
                                   LANES)] = entry4 + f
            pltpu.sync_copy(t2_hbm.at[idxb], valb)
            for l in range(NL2):
                fx, fy, fz = fracs[l]
                ws = _corner_weights(fx, fy, fz)
                for f in range(4):
                    s = ws[0] * valb[pl.ds(((l * 8) * 4 + f) * LANES, LANES)]
                    for c in range(1, 8):
                        s = s + ws[c] * valb[
                            pl.ds(((l * 8 + c) * 4 + f) * LANES, LANES)]
                    col = k * (NL2 * 4) + l * 4 + f
                    plsc.store_scatter(fbuf, [frow + col], s)

        pltpu.sync_copy(fbuf, feat_hbm.at[pl.ds((base + g16) * nfeat,
                                                LANES * nfeat)])


def _enc2_feat(xf, idxf, t2f, n_rays):
    info = plsc.get_sparse_core_info()
    nw = info.num_cores * info.num_subcores
    rpw = n_rays // nw
    mesh = plsc.VectorSubcoreMesh(core_axis_name="c", subcore_axis_name="s")
    body = functools.partial(_enc2_kernel_body, n_rays, rpw, info.num_cores)
    nfeat = 5 * NL2 * 4
    return pl.kernel(
        body,
        out_type=jax.ShapeDtypeStruct((n_rays * nfeat,), jnp.float32),
        mesh=mesh,
        compiler_params=pltpu.CompilerParams(needs_layout_passes=False),
        scratch_types=[
            pltpu.VMEM((4 * rpw,), jnp.float32),
            pltpu.VMEM((rpw,), jnp.int32),
            pltpu.VMEM((NL2 * 8 * 4 * LANES,), jnp.int32),
            pltpu.VMEM((NL2 * 8 * 4 * LANES,), jnp.float32),
            pltpu.VMEM((LANES * nfeat,), jnp.float32),
        ],
    )(xf, idxf, t2f)


def _mlp_kernel(f_ref, w1_ref, b1_ref, w2_ref, b2_ref, w3_ref, b3_ref,
                w4_ref, b4_ref, out_ref):
    h = jnp.maximum(
        jnp.dot(f_ref[...], w1_ref[...], preferred_element_type=jnp.float32)
        + b1_ref[...], 0.0)
    h = jnp.maximum(
        jnp.dot(h, w2_ref[...], preferred_element_type=jnp.float32)
        + b2_ref[...], 0.0)
    h = jnp.maximum(
        jnp.dot(h, w3_ref[...], preferred_element_type=jnp.float32)
        + b3_ref[...], 0.0)
    out_ref[...] = (
        jnp.dot(h, w4_ref[...], preferred_element_type=jnp.float32)
        + b4_ref[...])


def _mlp(feat, W1, b1, W2, b2, W3, b3, W4p, b4p, n_rays):
    blk = 256
    grid = n_rays // blk
    full = lambda shape: pl.BlockSpec(shape, lambda i: (0, 0))
    return pl.pallas_call(
        _mlp_kernel,
        grid=(grid,),
        in_specs=[
            pl.BlockSpec((blk, 120), lambda i: (i, 0)),
            full((120, 64)), full((1, 64)),
            full((64, 64)), full((1, 64)),
            full((64, 64)), full((1, 64)),
            full((64, 8)), full((1, 8)),
        ],
        out_specs=pl.BlockSpec((blk, 8), lambda i: (i, 0)),
        out_shape=jax.ShapeDtypeStruct((n_rays, 8), jnp.float32),
    )(feat, W1, b1.reshape(1, 64), W2, b2.reshape(1, 64),
      W3, b3.reshape(1, 64), W4p, b4p)


def kernel(x, table1, table2, Wl, bl, W1, b1, W2, b2, W3, b3, W4, b4):
    n_rays = x.shape[0]
    xf = x.T.reshape(4 * n_rays)
    t1f = table1.reshape(NL1 * HS)
    t2f = table2.reshape(NL2 * HS * 4)
    wlb = jnp.broadcast_to(Wl.reshape(NL1, 1), (NL1, LANES)).reshape(-1)
    blb = jnp.broadcast_to(bl.reshape(1, 1), (1, LANES)).reshape(-1)

    z = _enc1_logits(xf, t1f, wlb, blb, n_rays).reshape(n_rays, NPTS)
    label, hits, idxf = _labels(z, n_rays)
    feat = _enc2_feat(xf, idxf.reshape(n_rays), t2f, n_rays)
    feat = feat.reshape(n_rays, 5 * NL2 * 4)

    W4p = jnp.concatenate([W4, jnp.zeros((64, 5), W4.dtype)], axis=1)
    b4p = jnp.concatenate([b4, jnp.zeros((5,), b4.dtype)]).reshape(1, 8)
    rgb8 = _mlp(feat, W1, b1, W2, b2, W3, b3, W4p, b4p, n_rays)

    return (hits, label.reshape(n_rays, NPTS, 1), rgb8[:, :3])


# speedup vs baseline: 8.5943x; 2.5877x over previous
import functools

import jax
import jax.numpy as jnp
from jax import lax
from jax.experimental import pallas as pl
from jax.experimental.pallas import tpu as pltpu
from jax.experimental.pallas import tpu_sc as plsc

HS = 2 ** 19
MASK = HS - 1
NPTS = 200
C1 = 2654435761
C2 = 805459861
NL1 = 5
NL2 = 6
LANES = 16


def _hash_corner(ix, iy_m, iz_m):
    h = ix.astype(jnp.uint32) ^ iy_m ^ iz_m
    return (h & jnp.uint32(MASK)).astype(jnp.int32)


def _coords(ox, oy, dx, dy, tv):
    px = jnp.clip(ox + tv * dx, 0.0, 1.0)
    py = jnp.clip(oy + tv * dy, 0.0, 1.0)
    return px, py, tv


def _cell(p, res):
    pos = p * res
    p0 = pos.astype(jnp.int32)
    frac = pos - p0.astype(jnp.float32)
    return p0, frac


def _corner_weights(fx, fy, fz):
    wx0, wy0, wz0 = 1.0 - fx, 1.0 - fy, 1.0 - fz
    ws = []
    for wx in (wx0, fx):
        for wy in (wy0, fy):
            for wz in (wz0, fz):
                ws.append((wx * wy) * wz)
    return ws


def _corner_hashes(px, py, pz, res):
    x0, fx = _cell(px, res)
    y0, fy = _cell(py, res)
    z0, fz = _cell(pz, res)
    ym0 = y0.astype(jnp.uint32) * jnp.uint32(C1)
    ym1 = (y0 + 1).astype(jnp.uint32) * jnp.uint32(C1)
    zm0 = z0.astype(jnp.uint32) * jnp.uint32(C2)
    zm1 = (z0 + 1).astype(jnp.uint32) * jnp.uint32(C2)
    hs = []
    for xx in (x0, x0 + 1):
        for ym in (ym0, ym1):
            for zm in (zm0, zm1):
                hs.append(_hash_corner(xx, ym, zm))
    return hs, fx, fy, fz


JB = 4
NIDX1 = NL1 * 8 * LANES


def _enc1_kernel_body(n_rays, rpw, num_cores,
                      xf_hbm, wlb_hbm, blb_hbm, t1_hbm, z_hbm,
                      xv, wlv, blv, idxA, idxB, valA, valB, zbuf,
                      semA, semB):
    wid = lax.axis_index("s") * num_cores + lax.axis_index("c")
    base = wid * rpw
    for r in range(4):
        pltpu.sync_copy(xf_hbm.at[pl.ds(r * n_rays + base, rpw)],
                        xv.at[pl.ds(r * rpw, rpw)])
    pltpu.sync_copy(wlb_hbm, wlv)
    pltpu.sync_copy(blb_hbm, blv)
    iota = lax.iota(jnp.int32, LANES)
    n_groups = rpw // LANES

    @pl.loop(0, n_groups)
    def _group(g):
        g16 = g * LANES
        ox = xv[pl.ds(0 * rpw + g16, LANES)]
        oy = xv[pl.ds(1 * rpw + g16, LANES)]
        dx = xv[pl.ds(2 * rpw + g16, LANES)]
        dy = xv[pl.ds(3 * rpw + g16, LANES)]
        blvec = blv[...]
        zrow = iota * NPTS

        def build(jb, idxb):
            for u in range(JB):
                j = jnp.asarray(jb + u, jnp.float32)
                tv = jnp.full((LANES,), 1.0, jnp.float32) * (j * (1.0 / 199.0))
                px, py, pz = _coords(ox, oy, dx, dy, tv)
                for l in range(NL1):
                    hs, _, _, _ = _corner_hashes(px, py, pz, float(8 << l))
                    for c in range(8):
                        idxb[pl.ds((u * NL1 * 8 + l * 8 + c) * LANES,
                                   LANES)] = hs[c] + (l * HS)

        def start(idxb, valb, sem):
            return pltpu.async_copy(t1_hbm.at[idxb], valb, sem)

        def wait(idxb, valb, sem):
            pltpu.make_async_copy(t1_hbm.at[idxb], valb, sem).wait()

        def compute(jb, valb):
            for u in range(JB):
                j = jb + u
                jf = jnp.asarray(j, jnp.float32)
                tv = jnp.full((LANES,), 1.0, jnp.float32) * (jf * (1.0 / 199.0))
                px, py, pz = _coords(ox, oy, dx, dy, tv)
                z = blvec
                for l in range(NL1):
                    _, fx = _cell(px, float(8 << l))
                    _, fy = _cell(py, float(8 << l))
                    _, fz = _cell(pz, float(8 << l))
                    ws = _corner_weights(fx, fy, fz)
                    s = ws[0] * valb[pl.ds((u * NL1 * 8 + l * 8) * LANES,
                                           LANES)]
                    for c in range(1, 8):
                        s = s + ws[c] * valb[
                            pl.ds((u * NL1 * 8 + l * 8 + c) * LANES, LANES)]
                    z = z + s * wlv[pl.ds(l * LANES, LANES)]
                plsc.store_scatter(zbuf, [zrow + j], z)

        build(0, idxA)
        start(idxA, valA, semA)
        n_iter = NPTS // (2 * JB)

        @pl.loop(0, n_iter)
        def _pair(i):
            jb = i * (2 * JB)
            build(jb + JB, idxB)
            start(idxB, valB, semB)
            wait(idxA, valA, semA)
            compute(jb, valA)

            @pl.when(i < n_iter - 1)
            def _prefetch():
                build(jb + 2 * JB, idxA)
                start(idxA, valA, semA)

            wait(idxB, valB, semB)
            compute(jb + JB, valB)

        pltpu.sync_copy(zbuf, z_hbm.at[pl.ds((base + g16) * NPTS,
                                             LANES * NPTS)])


def _enc1_logits(xf, t1f, wlb, blb, n_rays):
    info = plsc.get_sparse_core_info()
    nw = info.num_cores * info.num_subcores
    rpw = n_rays // nw
    mesh = plsc.VectorSubcoreMesh(core_axis_name="c", subcore_axis_name="s")
    body = functools.partial(_enc1_kernel_body, n_rays, rpw, info.num_cores)
    return pl.kernel(
        body,
        out_type=jax.ShapeDtypeStruct((n_rays * NPTS,), jnp.float32),
        mesh=mesh,
        compiler_params=pltpu.CompilerParams(needs_layout_passes=False),
        scratch_types=[
            pltpu.VMEM((4 * rpw,), jnp.float32),
            pltpu.VMEM((NL1 * LANES,), jnp.float32),
            pltpu.VMEM((LANES,), jnp.float32),
            pltpu.VMEM((JB * NIDX1,), jnp.int32),
            pltpu.VMEM((JB * NIDX1,), jnp.int32),
            pltpu.VMEM((JB * NIDX1,), jnp.float32),
            pltpu.VMEM((JB * NIDX1,), jnp.float32),
            pltpu.VMEM((LANES * NPTS,), jnp.float32),
            pltpu.SemaphoreType.DMA,
            pltpu.SemaphoreType.DMA,
        ],
    )(xf, wlb, blb, t1f)


def _label_kernel(z_ref, lab_ref, hits_ref, idx_ref):
    z = z_ref[...]
    lab = 1.0 / (1.0 + jnp.exp(-z))
    lab_ref[...] = lab
    hits_ref[...] = jnp.max(lab, axis=1, keepdims=True)
    cond = lab > 0.5
    jmat = lax.broadcasted_iota(jnp.int32, z.shape, 1)
    masked = jnp.where(cond, jmat, NPTS)
    first = jnp.min(masked, axis=1, keepdims=True)
    idx_ref[...] = jnp.where(first == NPTS, 0, first)


def _labels(z, n_rays):
    blk = 256
    grid = n_rays // blk
    return pl.pallas_call(
        _label_kernel,
        grid=(grid,),
        in_specs=[pl.BlockSpec((blk, NPTS), lambda i: (i, 0))],
        out_specs=[
            pl.BlockSpec((blk, NPTS), lambda i: (i, 0)),
            pl.BlockSpec((blk, 1), lambda i: (i, 0)),
            pl.BlockSpec((blk, 1), lambda i: (i, 0)),
        ],
        out_shape=[
            jax.ShapeDtypeStruct((n_rays, NPTS), jnp.float32),
            jax.ShapeDtypeStruct((n_rays, 1), jnp.float32),
            jax.ShapeDtypeStruct((n_rays, 1), jnp.int32),
        ],
    )(z)


def _enc2_kernel_body(n_rays, rpw, num_cores,
                      xf_hbm, idx_hbm, t2_hbm, feat_hbm,
                      xv, iv, idxb, valb, fbuf):
    wid = lax.axis_index("s") * num_cores + lax.axis_index("c")
    base = wid * rpw
    for r in range(4):
        pltpu.sync_copy(xf_hbm.at[pl.ds(r * n_rays + base, rpw)],
                        xv.at[pl.ds(r * rpw, rpw)])
    pltpu.sync_copy(idx_hbm.at[pl.ds(base, rpw)], iv)
    iota = lax.iota(jnp.int32, LANES)
    n_groups = rpw // LANES
    nfeat = 5 * NL2 * 4

    @pl.loop(0, n_groups)
    def _group(g):
        g16 = g * LANES
        ox = xv[pl.ds(0 * rpw + g16, LANES)]
        oy = xv[pl.ds(1 * rpw + g16, LANES)]
        dx = xv[pl.ds(2 * rpw + g16, LANES)]
        dy = xv[pl.ds(3 * rpw + g16, LANES)]
        jfirst = iv[pl.ds(g16, LANES)]
        frow = iota * nfeat

        @pl.loop(0, 5)
        def _neighbor(k):
            jk = jnp.clip(jfirst + (k - 2), 0, NPTS - 1)
            tv = jk.astype(jnp.float32) * (1.0 / 199.0)
            px, py, pz = _coords(ox, oy, dx, dy, tv)
            fracs = []
            for l in range(NL2):
                hs, fx, fy, fz = _corner_hashes(px, py, pz, float(4 << l))
                fracs.append((fx, fy, fz))
                for c in range(8):
                    h = hs[c]
                    base = l * (HS * 4) + ((h >> 7) << 9) + (h & 127)
                    for f in range(4):
                        idxb[pl.ds(((l * 8 + c) * 4 + f) * LANES,
                                   LANES)] = base + f * 128
            pltpu.sync_copy(t2_hbm.at[idxb], valb)
            for l in range(NL2):
                fx, fy, fz = fracs[l]
                ws = _corner_weights(fx, fy, fz)
                for f in range(4):
                    s = ws[0] * valb[pl.ds(((l * 8) * 4 + f) * LANES, LANES)]
                    for c in range(1, 8):
                        s = s + ws[c] * valb[
                            pl.ds(((l * 8 + c) * 4 + f) * LANES, LANES)]
                    col = k * (NL2 * 4) + l * 4 + f
                    plsc.store_scatter(fbuf, [frow + col], s)

        pltpu.sync_copy(fbuf, feat_hbm.at[pl.ds((base + g16) * nfeat,
                                                LANES * nfeat)])


def _enc2_feat(xf, idxf, t2f, n_rays):
    info = plsc.get_sparse_core_info()
    nw = info.num_cores * info.num_subcores
    rpw = n_rays // nw
    mesh = plsc.VectorSubcoreMesh(core_axis_name="c", subcore_axis_name="s")
    body = functools.partial(_enc2_kernel_body, n_rays, rpw, info.num_cores)
    nfeat = 5 * NL2 * 4
    return pl.kernel(
        body,
        out_type=jax.ShapeDtypeStruct((n_rays * nfeat,), jnp.float32),
        mesh=mesh,
        compiler_params=pltpu.CompilerParams(needs_layout_passes=False),
        scratch_types=[
            pltpu.VMEM((4 * rpw,), jnp.float32),
            pltpu.VMEM((rpw,), jnp.int32),
            pltpu.VMEM((NL2 * 8 * 4 * LANES,), jnp.int32),
            pltpu.VMEM((NL2 * 8 * 4 * LANES,), jnp.float32),
            pltpu.VMEM((LANES * nfeat,), jnp.float32),
        ],
    )(xf, idxf, t2f)


def _mlp_kernel(f_ref, w1_ref, b1_ref, w2_ref, b2_ref, w3_ref, b3_ref,
                w4_ref, b4_ref, out_ref):
    h = jnp.maximum(
        jnp.dot(f_ref[...], w1_ref[...], preferred_element_type=jnp.float32)
        + b1_ref[...], 0.0)
    h = jnp.maximum(
        jnp.dot(h, w2_ref[...], preferred_element_type=jnp.float32)
        + b2_ref[...], 0.0)
    h = jnp.maximum(
        jnp.dot(h, w3_ref[...], preferred_element_type=jnp.float32)
        + b3_ref[...], 0.0)
    out_ref[...] = (
        jnp.dot(h, w4_ref[...], preferred_element_type=jnp.float32)
        + b4_ref[...])


def _mlp(feat, W1, b1, W2, b2, W3, b3, W4p, b4p, n_rays):
    blk = 256
    grid = n_rays // blk
    full = lambda shape: pl.BlockSpec(shape, lambda i: (0, 0))
    return pl.pallas_call(
        _mlp_kernel,
        grid=(grid,),
        in_specs=[
            pl.BlockSpec((blk, 120), lambda i: (i, 0)),
            full((120, 64)), full((1, 64)),
            full((64, 64)), full((1, 64)),
            full((64, 64)), full((1, 64)),
            full((64, 8)), full((1, 8)),
        ],
        out_specs=pl.BlockSpec((blk, 8), lambda i: (i, 0)),
        out_shape=jax.ShapeDtypeStruct((n_rays, 8), jnp.float32),
    )(feat, W1, b1.reshape(1, 64), W2, b2.reshape(1, 64),
      W3, b3.reshape(1, 64), W4p, b4p)


def kernel(x, table1, table2, Wl, bl, W1, b1, W2, b2, W3, b3, W4, b4):
    n_rays = x.shape[0]
    xf = x.T.reshape(4 * n_rays)
    t1f = table1.reshape(NL1 * HS)
    t2f = (table2.reshape(NL2, HS // 128, 128, 4)
           .transpose(0, 1, 3, 2).reshape(NL2 * HS * 4))
    wlb = jnp.broadcast_to(Wl.reshape(NL1, 1), (NL1, LANES)).reshape(-1)
    blb = jnp.broadcast_to(bl.reshape(1, 1), (1, LANES)).reshape(-1)

    z = _enc1_logits(xf, t1f, wlb, blb, n_rays).reshape(n_rays, NPTS)
    label, hits, idxf = _labels(z, n_rays)
    feat = _enc2_feat(xf, idxf.reshape(n_rays), t2f, n_rays)
    feat = feat.reshape(n_rays, 5 * NL2 * 4)

    W4p = jnp.concatenate([W4, jnp.zeros((64, 5), W4.dtype)], axis=1)
    b4p = jnp.concatenate([b4, jnp.zeros((5,), b4.dtype)]).reshape(1, 8)
    rgb8 = _mlp(feat, W1, b1, W2, b2, W3, b3, W4p, b4p, n_rays)

    return (hits, label.reshape(n_rays, NPTS, 1), rgb8[:, :3])


# speedup vs baseline: 8.6208x; 1.0031x over previous
import functools

import jax
import jax.numpy as jnp
from jax import lax
from jax.experimental import pallas as pl
from jax.experimental.pallas import tpu as pltpu
from jax.experimental.pallas import tpu_sc as plsc

HS = 2 ** 19
MASK = HS - 1
NPTS = 200
C1 = 2654435761
C2 = 805459861
NL1 = 5
NL2 = 6
LANES = 16


def _hash_corner(ix, iy_m, iz_m):
    h = ix.astype(jnp.uint32) ^ iy_m ^ iz_m
    return (h & jnp.uint32(MASK)).astype(jnp.int32)


def _coords(ox, oy, dx, dy, tv):
    px = jnp.clip(ox + tv * dx, 0.0, 1.0)
    py = jnp.clip(oy + tv * dy, 0.0, 1.0)
    return px, py, tv


def _cell(p, res):
    pos = p * res
    p0 = pos.astype(jnp.int32)
    frac = pos - p0.astype(jnp.float32)
    return p0, frac


def _corner_weights(fx, fy, fz):
    wx0, wy0, wz0 = 1.0 - fx, 1.0 - fy, 1.0 - fz
    ws = []
    for wx in (wx0, fx):
        for wy in (wy0, fy):
            for wz in (wz0, fz):
                ws.append((wx * wy) * wz)
    return ws


def _corner_hashes(px, py, pz, res):
    x0, fx = _cell(px, res)
    y0, fy = _cell(py, res)
    z0, fz = _cell(pz, res)
    ym0 = y0.astype(jnp.uint32) * jnp.uint32(C1)
    ym1 = (y0 + 1).astype(jnp.uint32) * jnp.uint32(C1)
    zm0 = z0.astype(jnp.uint32) * jnp.uint32(C2)
    zm1 = (z0 + 1).astype(jnp.uint32) * jnp.uint32(C2)
    hs = []
    for xx in (x0, x0 + 1):
        for ym in (ym0, ym1):
            for zm in (zm0, zm1):
                hs.append(_hash_corner(xx, ym, zm))
    return hs, fx, fy, fz


JB = 10
NIDX1 = NL1 * 8 * LANES


def _enc1_kernel_body(n_rays, rpw, num_cores,
                      xf_hbm, wlb_hbm, blb_hbm, t1_hbm, z_hbm,
                      xv, wlv, blv, idxA, idxB, valA, valB, zbuf,
                      semA, semB):
    wid = lax.axis_index("s") * num_cores + lax.axis_index("c")
    base = wid * rpw
    for r in range(4):
        pltpu.sync_copy(xf_hbm.at[pl.ds(r * n_rays + base, rpw)],
                        xv.at[pl.ds(r * rpw, rpw)])
    pltpu.sync_copy(wlb_hbm, wlv)
    pltpu.sync_copy(blb_hbm, blv)
    iota = lax.iota(jnp.int32, LANES)
    n_groups = rpw // LANES

    @pl.loop(0, n_groups)
    def _group(g):
        g16 = g * LANES
        ox = xv[pl.ds(0 * rpw + g16, LANES)]
        oy = xv[pl.ds(1 * rpw + g16, LANES)]
        dx = xv[pl.ds(2 * rpw + g16, LANES)]
        dy = xv[pl.ds(3 * rpw + g16, LANES)]
        blvec = blv[...]
        zrow = iota * NPTS

        def build(jb, idxb):
            for u in range(JB):
                j = jnp.asarray(jb + u, jnp.float32)
                tv = jnp.full((LANES,), 1.0, jnp.float32) * (j * (1.0 / 199.0))
                px, py, pz = _coords(ox, oy, dx, dy, tv)
                for l in range(NL1):
                    hs, _, _, _ = _corner_hashes(px, py, pz, float(8 << l))
                    for c in range(8):
                        idxb[pl.ds((u * NL1 * 8 + l * 8 + c) * LANES,
                                   LANES)] = hs[c] + (l * HS)

        def start(idxb, valb, sem):
            return pltpu.async_copy(t1_hbm.at[idxb], valb, sem)

        def wait(idxb, valb, sem):
            pltpu.make_async_copy(t1_hbm.at[idxb], valb, sem).wait()

        def compute(jb, valb):
            for u in range(JB):
                j = jb + u
                jf = jnp.asarray(j, jnp.float32)
                tv = jnp.full((LANES,), 1.0, jnp.float32) * (jf * (1.0 / 199.0))
                px, py, pz = _coords(ox, oy, dx, dy, tv)
                z = blvec
                for l in range(NL1):
                    _, fx = _cell(px, float(8 << l))
                    _, fy = _cell(py, float(8 << l))
                    _, fz = _cell(pz, float(8 << l))
                    ws = _corner_weights(fx, fy, fz)
                    s = ws[0] * valb[pl.ds((u * NL1 * 8 + l * 8) * LANES,
                                           LANES)]
                    for c in range(1, 8):
                        s = s + ws[c] * valb[
                            pl.ds((u * NL1 * 8 + l * 8 + c) * LANES, LANES)]
                    z = z + s * wlv[pl.ds(l * LANES, LANES)]
                plsc.store_scatter(zbuf, [zrow + j], z)

        build(0, idxA)
        start(idxA, valA, semA)
        n_iter = NPTS // (2 * JB)

        @pl.loop(0, n_iter)
        def _pair(i):
            jb = i * (2 * JB)
            build(jb + JB, idxB)
            start(idxB, valB, semB)
            wait(idxA, valA, semA)
            compute(jb, valA)

            @pl.when(i < n_iter - 1)
            def _prefetch():
                build(jb + 2 * JB, idxA)
                start(idxA, valA, semA)

            wait(idxB, valB, semB)
            compute(jb + JB, valB)

        pltpu.sync_copy(zbuf, z_hbm.at[pl.ds((base + g16) * NPTS,
                                             LANES * NPTS)])


def _enc1_logits(xf, t1f, wlb, blb, n_rays):
    info = plsc.get_sparse_core_info()
    nw = info.num_cores * info.num_subcores
    rpw = n_rays // nw
    mesh = plsc.VectorSubcoreMesh(core_axis_name="c", subcore_axis_name="s")
    body = functools.partial(_enc1_kernel_body, n_rays, rpw, info.num_cores)
    return pl.kernel(
        body,
        out_type=jax.ShapeDtypeStruct((n_rays * NPTS,), jnp.float32),
        mesh=mesh,
        compiler_params=pltpu.CompilerParams(needs_layout_passes=False),
        scratch_types=[
            pltpu.VMEM((4 * rpw,), jnp.float32),
            pltpu.VMEM((NL1 * LANES,), jnp.float32),
            pltpu.VMEM((LANES,), jnp.float32),
            pltpu.VMEM((JB * NIDX1,), jnp.int32),
            pltpu.VMEM((JB * NIDX1,), jnp.int32),
            pltpu.VMEM((JB * NIDX1,), jnp.float32),
            pltpu.VMEM((JB * NIDX1,), jnp.float32),
            pltpu.VMEM((LANES * NPTS,), jnp.float32),
            pltpu.SemaphoreType.DMA,
            pltpu.SemaphoreType.DMA,
        ],
    )(xf, wlb, blb, t1f)


def _label_kernel(z_ref, lab_ref, hits_ref, idx_ref):
    z = z_ref[...]
    lab = 1.0 / (1.0 + jnp.exp(-z))
    lab_ref[...] = lab
    hits_ref[...] = jnp.max(lab, axis=1, keepdims=True)
    cond = lab > 0.5
    jmat = lax.broadcasted_iota(jnp.int32, z.shape, 1)
    masked = jnp.where(cond, jmat, NPTS)
    first = jnp.min(masked, axis=1, keepdims=True)
    idx_ref[...] = jnp.where(first == NPTS, 0, first)


def _labels(z, n_rays):
    blk = 256
    grid = n_rays // blk
    return pl.pallas_call(
        _label_kernel,
        grid=(grid,),
        in_specs=[pl.BlockSpec((blk, NPTS), lambda i: (i, 0))],
        out_specs=[
            pl.BlockSpec((blk, NPTS), lambda i: (i, 0)),
            pl.BlockSpec((blk, 1), lambda i: (i, 0)),
            pl.BlockSpec((blk, 1), lambda i: (i, 0)),
        ],
        out_shape=[
            jax.ShapeDtypeStruct((n_rays, NPTS), jnp.float32),
            jax.ShapeDtypeStruct((n_rays, 1), jnp.float32),
            jax.ShapeDtypeStruct((n_rays, 1), jnp.int32),
        ],
    )(z)


def _enc2_kernel_body(n_rays, rpw, num_cores,
                      xf_hbm, idx_hbm, t2_hbm, feat_hbm,
                      xv, iv, idxb, valb, fbuf):
    wid = lax.axis_index("s") * num_cores + lax.axis_index("c")
    base = wid * rpw
    for r in range(4):
        pltpu.sync_copy(xf_hbm.at[pl.ds(r * n_rays + base, rpw)],
                        xv.at[pl.ds(r * rpw, rpw)])
    pltpu.sync_copy(idx_hbm.at[pl.ds(base, rpw)], iv)
    iota = lax.iota(jnp.int32, LANES)
    n_groups = rpw // LANES
    nfeat = 5 * NL2 * 4

    @pl.loop(0, n_groups)
    def _group(g):
        g16 = g * LANES
        ox = xv[pl.ds(0 * rpw + g16, LANES)]
        oy = xv[pl.ds(1 * rpw + g16, LANES)]
        dx = xv[pl.ds(2 * rpw + g16, LANES)]
        dy = xv[pl.ds(3 * rpw + g16, LANES)]
        jfirst = iv[pl.ds(g16, LANES)]
        frow = iota * nfeat

        @pl.loop(0, 5)
        def _neighbor(k):
            jk = jnp.clip(jfirst + (k - 2), 0, NPTS - 1)
            tv = jk.astype(jnp.float32) * (1.0 / 199.0)
            px, py, pz = _coords(ox, oy, dx, dy, tv)
            fracs = []
            for l in range(NL2):
                hs, fx, fy, fz = _corner_hashes(px, py, pz, float(4 << l))
                fracs.append((fx, fy, fz))
                for c in range(8):
                    h = hs[c]
                    base = l * (HS * 4) + ((h >> 7) << 9) + (h & 127)
                    for f in range(4):
                        idxb[pl.ds(((l * 8 + c) * 4 + f) * LANES,
                                   LANES)] = base + f * 128
            pltpu.sync_copy(t2_hbm.at[idxb], valb)
            for l in range(NL2):
                fx, fy, fz = fracs[l]
                ws = _corner_weights(fx, fy, fz)
                for f in range(4):
                    s = ws[0] * valb[pl.ds(((l * 8) * 4 + f) * LANES, LANES)]
                    for c in range(1, 8):
                        s = s + ws[c] * valb[
                            pl.ds(((l * 8 + c) * 4 + f) * LANES, LANES)]
                    col = k * (NL2 * 4) + l * 4 + f
                    plsc.store_scatter(fbuf, [frow + col], s)

        pltpu.sync_copy(fbuf, feat_hbm.at[pl.ds((base + g16) * nfeat,
                                                LANES * nfeat)])


def _enc2_feat(xf, idxf, t2f, n_rays):
    info = plsc.get_sparse_core_info()
    nw = info.num_cores * info.num_subcores
    rpw = n_rays // nw
    mesh = plsc.VectorSubcoreMesh(core_axis_name="c", subcore_axis_name="s")
    body = functools.partial(_enc2_kernel_body, n_rays, rpw, info.num_cores)
    nfeat = 5 * NL2 * 4
    return pl.kernel(
        body,
        out_type=jax.ShapeDtypeStruct((n_rays * nfeat,), jnp.float32),
        mesh=mesh,
        compiler_params=pltpu.CompilerParams(needs_layout_passes=False),
        scratch_types=[
            pltpu.VMEM((4 * rpw,), jnp.float32),
            pltpu.VMEM((rpw,), jnp.int32),
            pltpu.VMEM((NL2 * 8 * 4 * LANES,), jnp.int32),
            pltpu.VMEM((NL2 * 8 * 4 * LANES,), jnp.float32),
            pltpu.VMEM((LANES * nfeat,), jnp.float32),
        ],
    )(xf, idxf, t2f)


def _mlp_kernel(f_ref, w1_ref, b1_ref, w2_ref, b2_ref, w3_ref, b3_ref,
                w4_ref, b4_ref, out_ref):
    h = jnp.maximum(
        jnp.dot(f_ref[...], w1_ref[...], preferred_element_type=jnp.float32)
        + b1_ref[...], 0.0)
    h = jnp.maximum(
        jnp.dot(h, w2_ref[...], preferred_element_type=jnp.float32)
        + b2_ref[...], 0.0)
    h = jnp.maximum(
        jnp.dot(h, w3_ref[...], preferred_element_type=jnp.float32)
        + b3_ref[...], 0.0)
    out_ref[...] = (
        jnp.dot(h, w4_ref[...], preferred_element_type=jnp.float32)
        + b4_ref[...])


def _mlp(feat, W1, b1, W2, b2, W3, b3, W4p, b4p, n_rays):
    blk = 256
    grid = n_rays // blk
    full = lambda shape: pl.BlockSpec(shape, lambda i: (0, 0))
    return pl.pallas_call(
        _mlp_kernel,
        grid=(grid,),
        in_specs=[
            pl.BlockSpec((blk, 120), lambda i: (i, 0)),
            full((120, 64)), full((1, 64)),
            full((64, 64)), full((1, 64)),
            full((64, 64)), full((1, 64)),
            full((64, 8)), full((1, 8)),
        ],
        out_specs=pl.BlockSpec((blk, 8), lambda i: (i, 0)),
        out_shape=jax.ShapeDtypeStruct((n_rays, 8), jnp.float32),
    )(feat, W1, b1.reshape(1, 64), W2, b2.reshape(1, 64),
      W3, b3.reshape(1, 64), W4p, b4p)


def kernel(x, table1, table2, Wl, bl, W1, b1, W2, b2, W3, b3, W4, b4):
    n_rays = x.shape[0]
    xf = x.T.reshape(4 * n_rays)
    t1f = table1.reshape(NL1 * HS)
    t2f = (table2.reshape(NL2, HS // 128, 128, 4)
           .transpose(0, 1, 3, 2).reshape(NL2 * HS * 4))
    wlb = jnp.broadcast_to(Wl.reshape(NL1, 1), (NL1, LANES)).reshape(-1)
    blb = jnp.broadcast_to(bl.reshape(1, 1), (1, LANES)).reshape(-1)

    z = _enc1_logits(xf, t1f, wlb, blb, n_rays).reshape(n_rays, NPTS)
    label, hits, idxf = _labels(z, n_rays)
    feat = _enc2_feat(xf, idxf.reshape(n_rays), t2f, n_rays)
    feat = feat.reshape(n_rays, 5 * NL2 * 4)

    W4p = jnp.concatenate([W4, jnp.zeros((64, 5), W4.dtype)], axis=1)
    b4p = jnp.concatenate([b4, jnp.zeros((5,), b4.dtype)]).reshape(1, 8)
    rgb8 = _mlp(feat, W1, b1, W2, b2, W3, b3, W4p, b4p, n_rays)

    return (hits, label.reshape(n_rays, NPTS, 1), rgb8[:, :3])
